# baseline (speedup 1.0000x reference)
import functools

import jax
import jax.numpy as jnp
from jax import lax
from jax.experimental import pallas as pl
from jax.experimental.pallas import tpu as pltpu
from jax.experimental.pallas import tpu_sc as plsc

_NC = 2
_NS = 16
_NW = _NC * _NS


_CHUNK_ROWS = 4


def _copy_body(rows_per_tile, chunks_per_layer, in_hbm, out_hbm,
               buf0, buf1, lsem0, lsem1, ssem0, ssem1):
    wid = lax.axis_index("s") * _NC + lax.axis_index("c")
    layer = wid // chunks_per_layer
    row0 = (wid % chunks_per_layer) * rows_per_tile

    bufs = (buf0, buf1)
    lsems = (lsem0, lsem1)
    ssems = (ssem0, ssem1)
    n_chunks = rows_per_tile // _CHUNK_ROWS
    store_handles = [None, None]
    for i in range(n_chunks):
        slot = i % 2
        r = row0 + i * _CHUNK_ROWS
        if store_handles[slot] is not None:
            store_handles[slot].wait()
        pltpu.async_copy(
            in_hbm.at[2 * layer + 1, pl.ds(r, _CHUNK_ROWS)],
            bufs[slot], lsems[slot],
        ).wait()
        store_handles[slot] = pltpu.async_copy(
            bufs[slot], out_hbm.at[layer, pl.ds(r, _CHUNK_ROWS)], ssems[slot],
        )
    for h in store_handles:
        if h is not None:
            h.wait()


def kernel(hidden_states):
    num_dirs_layers, batch, hidden = hidden_states.shape
    num_layers = num_dirs_layers // 2
    chunks_per_layer = _NW // num_layers
    rows_per_tile = batch // chunks_per_layer

    mesh = plsc.VectorSubcoreMesh(core_axis_name="c", subcore_axis_name="s")
    return pl.kernel(
        functools.partial(_copy_body, rows_per_tile, chunks_per_layer),
        mesh=mesh,
        out_type=jax.ShapeDtypeStruct((num_layers, batch, hidden), jnp.float32),
        scratch_types=[
            pltpu.VMEM((_CHUNK_ROWS, hidden), jnp.float32),
            pltpu.VMEM((_CHUNK_ROWS, hidden), jnp.float32),
            pltpu.SemaphoreType.DMA,
            pltpu.SemaphoreType.DMA,
            pltpu.SemaphoreType.DMA,
            pltpu.SemaphoreType.DMA,
        ],
    )(hidden_states)


# speedup vs baseline: 1.0667x; 1.0667x over previous
import functools

import jax
import jax.numpy as jnp
from jax import lax
from jax.experimental import pallas as pl
from jax.experimental.pallas import tpu as pltpu
from jax.experimental.pallas import tpu_sc as plsc

_NC = 2
_NS = 16
_NW = _NC * _NS


def _copy_body(rows_per_tile, chunks_per_layer, in_hbm, out_hbm,
               buf0, buf1, lsem0, lsem1, ssem0, ssem1):
    wid = lax.axis_index("s") * _NC + lax.axis_index("c")
    layer = wid // chunks_per_layer
    row0 = (wid % chunks_per_layer) * rows_per_tile

    half = rows_per_tile // 2
    in_layer = in_hbm.at[2 * layer + 1]
    out_layer = out_hbm.at[layer]
    l0 = pltpu.async_copy(in_layer.at[pl.ds(row0, half)], buf0, lsem0)
    l1 = pltpu.async_copy(in_layer.at[pl.ds(row0 + half, half)], buf1, lsem1)
    l0.wait()
    s0 = pltpu.async_copy(buf0, out_layer.at[pl.ds(row0, half)], ssem0)
    l1.wait()
    s1 = pltpu.async_copy(buf1, out_layer.at[pl.ds(row0 + half, half)], ssem1)
    s0.wait()
    s1.wait()


def kernel(hidden_states):
    num_dirs_layers, batch, hidden = hidden_states.shape
    num_layers = num_dirs_layers // 2
    chunks_per_layer = _NW // num_layers
    rows_per_tile = batch // chunks_per_layer

    mesh = plsc.VectorSubcoreMesh(core_axis_name="c", subcore_axis_name="s")
    return pl.kernel(
        functools.partial(_copy_body, rows_per_tile, chunks_per_layer),
        mesh=mesh,
        out_type=jax.ShapeDtypeStruct((num_layers, batch, hidden), jnp.float32),
        scratch_types=[
            pltpu.VMEM((batch // chunks_per_layer // 2, hidden), jnp.float32),
            pltpu.VMEM((batch // chunks_per_layer // 2, hidden), jnp.float32),
            pltpu.SemaphoreType.DMA,
            pltpu.SemaphoreType.DMA,
            pltpu.SemaphoreType.DMA,
            pltpu.SemaphoreType.DMA,
        ],
    )(hidden_states)
